# Initial kernel scaffold; baseline (speedup 1.0000x reference)
import functools

import jax
import jax.numpy as jnp
from jax import lax
from jax.experimental import pallas as pl
from jax.experimental.pallas import tpu as pltpu
from jax.experimental.pallas import tpu_sc as plsc

_G = 512
_F = 16
_NC = 2
_NS = 16
_EBLK = 80
_ZR = 625
_BN = 5000
_BE = 20000


def _embed_nodes(x, WnA, WnB, bnA, bnB):
    n, d = x.shape
    nb = n // _BN

    def body(x_ref, wa, wb, ba, bb, oa, ob):
        xb = x_ref[...]
        oa[...] = jnp.dot(xb, wa[...], preferred_element_type=jnp.float32) + ba[...]
        ob[...] = jnp.dot(xb, wb[...], preferred_element_type=jnp.float32) + bb[...]

    return pl.pallas_call(
        body,
        grid=(nb,),
        in_specs=[
            pl.BlockSpec((_BN, d), lambda i: (i, 0)),
            pl.BlockSpec(WnA.shape, lambda i: (0, 0)),
            pl.BlockSpec(WnB.shape, lambda i: (0, 0)),
            pl.BlockSpec((1, _F), lambda i: (0, 0)),
            pl.BlockSpec((1, _F), lambda i: (0, 0)),
        ],
        out_specs=[
            pl.BlockSpec((_BN, _F), lambda i: (i, 0)),
            pl.BlockSpec((_BN, _F), lambda i: (i, 0)),
        ],
        out_shape=[
            jax.ShapeDtypeStruct((n, _F), jnp.float32),
            jax.ShapeDtypeStruct((n, _F), jnp.float32),
        ],
    )(x, WnA, WnB, bnA, bnB)


def _embed_edges(edge_attr, WeA, WeB, beA, beB):
    e, d = edge_attr.shape
    nb = e // _BE

    def body(a_ref, wa, wb, ba, bb, oa, ob):
        a = a_ref[...]
        oa[...] = (ba[...] + a[:, 0:1] * wa[0:1, :] + a[:, 1:2] * wa[1:2, :]
                   + a[:, 2:3] * wa[2:3, :])
        ob[...] = (bb[...] + a[:, 0:1] * wb[0:1, :] + a[:, 1:2] * wb[1:2, :]
                   + a[:, 2:3] * wb[2:3, :])

    return pl.pallas_call(
        body,
        grid=(nb,),
        in_specs=[
            pl.BlockSpec((_BE, d), lambda i: (i, 0)),
            pl.BlockSpec(WeA.shape, lambda i: (0, 0)),
            pl.BlockSpec(WeB.shape, lambda i: (0, 0)),
            pl.BlockSpec((1, _F), lambda i: (0, 0)),
            pl.BlockSpec((1, _F), lambda i: (0, 0)),
        ],
        out_specs=[
            pl.BlockSpec((_BE, _F), lambda i: (i, 0)),
            pl.BlockSpec((_BE, _F), lambda i: (i, 0)),
        ],
        out_shape=[
            jax.ShapeDtypeStruct((e, _F), jnp.float32),
            jax.ShapeDtypeStruct((e, _F), jnp.float32),
        ],
    )(edge_attr, WeA, WeB, beA, beB)


def _sc_conv(hA, hB, eaA, eaB, src, dst):
    n = hA.shape[0]
    e = src.shape[0]
    ept = e // _NS
    nblk = ept // _EBLK
    rpt = n // _NS
    nz = rpt // _ZR
    mesh = plsc.VectorSubcoreMesh(core_axis_name="c", subcore_axis_name="s",
                                  num_cores=_NC, num_subcores=_NS)

    @functools.partial(
        pl.kernel,
        out_type=[
            jax.ShapeDtypeStruct((n, _F), jnp.float32),
            jax.ShapeDtypeStruct((n, _F), jnp.float32),
        ],
        mesh=mesh,
        scratch_types=[
            pltpu.VMEM((_EBLK,), jnp.int32),
            pltpu.VMEM((_EBLK,), jnp.int32),
            pltpu.VMEM((_EBLK, _F), jnp.float32),
            pltpu.VMEM((_EBLK, _F), jnp.float32),
            pltpu.VMEM((_ZR, _F), jnp.float32),
            pltpu.SemaphoreType.DMA,
            pltpu.SemaphoreType.DMA,
            pltpu.VMEM_SHARED((n, _F), jnp.float32),
        ],
    )
    def conv(hA_hbm, hB_hbm, eaA_hbm, eaB_hbm, src_hbm, dst_hbm,
             outA_hbm, outB_hbm,
             sidx, didx, hbuf, ebuf, zbuf, gsem, esem, acc):
        c = lax.axis_index("c")
        s = lax.axis_index("s")
        zv = jnp.zeros((_F,), jnp.float32)

        def zrow(i, carry):
            zbuf[i, :] = zv
            return carry

        lax.fori_loop(0, _ZR, zrow, 0, unroll=8)

        def zcopy(k, carry):
            pltpu.sync_copy(zbuf, acc.at[pl.ds(s * rpt + k * _ZR, _ZR)])
            return carry

        lax.fori_loop(0, nz, zcopy, 0)
        plsc.subcore_barrier()

        def run(h_hbm, ea_hbm):
            def blk(b, carry):
                base = s * ept + b * _EBLK
                pltpu.sync_copy(src_hbm.at[pl.ds(base, _EBLK)], sidx)
                pltpu.sync_copy(dst_hbm.at[pl.ds(base, _EBLK)], didx)
                cea = pltpu.async_copy(ea_hbm.at[pl.ds(base, _EBLK)], ebuf, esem)
                cg = pltpu.async_copy(h_hbm.at[sidx], hbuf, gsem)
                cea.wait()
                cg.wait()

                def row(i, carry2):
                    hbuf[i, :] = jnp.maximum(hbuf[i, :] + ebuf[i, :], 0.0)
                    return carry2

                lax.fori_loop(0, _EBLK, row, 0, unroll=8)
                pltpu.sync_copy(hbuf, acc.at[didx], add=True)
                return carry

            lax.fori_loop(0, nblk, blk, 0)

        @pl.when(c == 0)
        def _():
            run(hA_hbm, eaA_hbm)

        @pl.when(c == 1)
        def _():
            run(hB_hbm, eaB_hbm)

        plsc.subcore_barrier()

        def wb(out_hbm):
            pltpu.sync_copy(acc.at[pl.ds(s * rpt, rpt)],
                            out_hbm.at[pl.ds(s * rpt, rpt)])

        @pl.when(c == 0)
        def _():
            wb(outA_hbm)

        @pl.when(c == 1)
        def _():
            wb(outB_hbm)

    return conv(hA, hB, eaA, eaB, src, dst)


def _mlp_stats(hA, hB, agA, agB, W1A, W1B, b1, W2, b2):
    n = hA.shape[0]
    nb = n // _BN

    def body(ha, hb, aa, ab, w1a, w1b, b1r, w2, b2r, z_ref, s1_ref, s2_ref):
        i = pl.program_id(0)
        xa = ha[...] + aa[...]
        xb = hb[...] + ab[...]
        t = jnp.maximum(
            jnp.dot(xa, w1a[...], preferred_element_type=jnp.float32)
            + jnp.dot(xb, w1b[...], preferred_element_type=jnp.float32)
            + b1r[...], 0.0)
        z = jnp.dot(t, w2[...], preferred_element_type=jnp.float32) + b2r[...]
        z_ref[...] = z
        p1 = jnp.sum(z, axis=0, keepdims=True)
        p2 = jnp.sum(z * z, axis=0, keepdims=True)

        @pl.when(i == 0)
        def _():
            s1_ref[...] = p1
            s2_ref[...] = p2

        @pl.when(i > 0)
        def _():
            s1_ref[...] += p1
            s2_ref[...] += p2

    return pl.pallas_call(
        body,
        grid=(nb,),
        in_specs=[
            pl.BlockSpec((_BN, _F), lambda i: (i, 0)),
            pl.BlockSpec((_BN, _F), lambda i: (i, 0)),
            pl.BlockSpec((_BN, _F), lambda i: (i, 0)),
            pl.BlockSpec((_BN, _F), lambda i: (i, 0)),
            pl.BlockSpec(W1A.shape, lambda i: (0, 0)),
            pl.BlockSpec(W1B.shape, lambda i: (0, 0)),
            pl.BlockSpec(b1.shape, lambda i: (0, 0)),
            pl.BlockSpec(W2.shape, lambda i: (0, 0)),
            pl.BlockSpec(b2.shape, lambda i: (0, 0)),
        ],
        out_specs=[
            pl.BlockSpec((_BN, 2 * _F), lambda i: (i, 0)),
            pl.BlockSpec((1, 2 * _F), lambda i: (0, 0)),
            pl.BlockSpec((1, 2 * _F), lambda i: (0, 0)),
        ],
        out_shape=[
            jax.ShapeDtypeStruct((n, 2 * _F), jnp.float32),
            jax.ShapeDtypeStruct((1, 2 * _F), jnp.float32),
            jax.ShapeDtypeStruct((1, 2 * _F), jnp.float32),
        ],
    )(hA, hB, agA, agB, W1A, W1B, b1, W2, b2)


def _norm_relu(z, s1, s2, g, beta):
    n = z.shape[0]
    nb = n // _BN
    inv_n = 1.0 / n

    def body(z_ref, s1_ref, s2_ref, g_ref, b_ref, oa, ob):
        mu = s1_ref[...] * inv_n
        var = s2_ref[...] * inv_n - mu * mu
        scale = g_ref[...] * lax.rsqrt(var + 1e-5)
        h = jnp.maximum((z_ref[...] - mu) * scale + b_ref[...], 0.0)
        oa[...] = h[:, :_F]
        ob[...] = h[:, _F:]

    return pl.pallas_call(
        body,
        grid=(nb,),
        in_specs=[
            pl.BlockSpec((_BN, 2 * _F), lambda i: (i, 0)),
            pl.BlockSpec((1, 2 * _F), lambda i: (0, 0)),
            pl.BlockSpec((1, 2 * _F), lambda i: (0, 0)),
            pl.BlockSpec((1, 2 * _F), lambda i: (0, 0)),
            pl.BlockSpec((1, 2 * _F), lambda i: (0, 0)),
        ],
        out_specs=[
            pl.BlockSpec((_BN, _F), lambda i: (i, 0)),
            pl.BlockSpec((_BN, _F), lambda i: (i, 0)),
        ],
        out_shape=[
            jax.ShapeDtypeStruct((n, _F), jnp.float32),
            jax.ShapeDtypeStruct((n, _F), jnp.float32),
        ],
    )(z, s1, s2, g, beta)


def _pool(hA, hB, batch2d):
    n = hA.shape[0]
    nb = n // _BN

    def body(ha, hb, bt, sa_ref, sb_ref, c_ref):
        i = pl.program_id(0)
        ids = bt[...]
        lab = lax.broadcasted_iota(jnp.int32, (_BN, _G), 1)
        oh = (ids == lab).astype(jnp.float32)
        dn = (((0,), (0,)), ((), ()))
        pa = lax.dot_general(oh, ha[...], dn, preferred_element_type=jnp.float32)
        pb = lax.dot_general(oh, hb[...], dn, preferred_element_type=jnp.float32)
        ones = jnp.ones((_BN, 1), jnp.float32)
        pc = lax.dot_general(oh, ones, dn, preferred_element_type=jnp.float32)

        @pl.when(i == 0)
        def _():
            sa_ref[...] = pa
            sb_ref[...] = pb
            c_ref[...] = pc

        @pl.when(i > 0)
        def _():
            sa_ref[...] += pa
            sb_ref[...] += pb
            c_ref[...] += pc

    return pl.pallas_call(
        body,
        grid=(nb,),
        in_specs=[
            pl.BlockSpec((_BN, _F), lambda i: (i, 0)),
            pl.BlockSpec((_BN, _F), lambda i: (i, 0)),
            pl.BlockSpec((_BN, 1), lambda i: (i, 0)),
        ],
        out_specs=[
            pl.BlockSpec((_G, _F), lambda i: (0, 0)),
            pl.BlockSpec((_G, _F), lambda i: (0, 0)),
            pl.BlockSpec((_G, 1), lambda i: (0, 0)),
        ],
        out_shape=[
            jax.ShapeDtypeStruct((_G, _F), jnp.float32),
            jax.ShapeDtypeStruct((_G, _F), jnp.float32),
            jax.ShapeDtypeStruct((_G, 1), jnp.float32),
        ],
    )(hA, hB, batch2d)


def _head(sA, sB, cnt, Wl1A, Wl1B, bl1, Wl2, bl2):
    def body(sa, sb, c, w1a, w1b, b1r, w2, b2r, o_ref):
        inv = 1.0 / jnp.maximum(c[...], 1.0)
        ga = sa[...] * inv
        gb = sb[...] * inv
        t = jnp.maximum(
            jnp.dot(ga, w1a[...], preferred_element_type=jnp.float32)
            + jnp.dot(gb, w1b[...], preferred_element_type=jnp.float32)
            + b1r[...], 0.0)
        o_ref[...] = jnp.dot(t, w2[...], preferred_element_type=jnp.float32) + b2r[...]

    return pl.pallas_call(
        body,
        out_shape=jax.ShapeDtypeStruct((_G, 2), jnp.float32),
    )(sA, sB, cnt, Wl1A, Wl1B, bl1, Wl2, bl2)


def kernel(x, edge_index, edge_attr, batch, Wn, bn, We, be,
           W1_0, b1_0, W2_0, b2_0, g0, beta0,
           W1_1, b1_1, W2_1, b2_1, g1, beta1,
           Wl1, bl1, Wl2, bl2):
    src = edge_index[0]
    dst = edge_index[1]
    batch2d = batch.reshape(-1, 1)

    hA, hB = _embed_nodes(x, Wn[:, :_F], Wn[:, _F:],
                          bn[:_F].reshape(1, _F), bn[_F:].reshape(1, _F))
    eaA, eaB = _embed_edges(edge_attr, We[:, :_F], We[:, _F:],
                            be[:_F].reshape(1, _F), be[_F:].reshape(1, _F))

    agA, agB = _sc_conv(hA, hB, eaA, eaB, src, dst)
    z, s1, s2 = _mlp_stats(hA, hB, agA, agB, W1_0[:_F], W1_0[_F:],
                           b1_0.reshape(1, -1), W2_0, b2_0.reshape(1, -1))
    h1A, h1B = _norm_relu(z, s1, s2, g0.reshape(1, -1), beta0.reshape(1, -1))

    agA, agB = _sc_conv(h1A, h1B, eaA, eaB, src, dst)
    z, s1, s2 = _mlp_stats(h1A, h1B, agA, agB, W1_1[:_F], W1_1[_F:],
                           b1_1.reshape(1, -1), W2_1, b2_1.reshape(1, -1))
    h2A, h2B = _norm_relu(z, s1, s2, g1.reshape(1, -1), beta1.reshape(1, -1))

    sA, sB, cnt = _pool(h2A, h2B, batch2d)
    return _head(sA, sB, cnt, Wl1[:_F], Wl1[_F:],
                 bl1.reshape(1, -1), Wl2, bl2.reshape(1, -1))


# speedup vs baseline: 2.0137x; 2.0137x over previous
import functools

import jax
import jax.numpy as jnp
from jax import lax
from jax.experimental import pallas as pl
from jax.experimental.pallas import tpu as pltpu
from jax.experimental.pallas import tpu_sc as plsc

_G = 512
_F = 16
_NC = 2
_NS = 16
_EBLK = 80
_ZR = 400
_WB = 800
_BN = 2000
_BE = 4000


def _embed_nodes(x, WnA, WnB, bnA, bnB):
    n, d = x.shape
    nb = n // _BN

    def body(x_ref, wa, wb, ba, bb, oa, ob):
        xb = x_ref[...]
        oa[...] = jnp.dot(xb, wa[...], preferred_element_type=jnp.float32) + ba[...]
        ob[...] = jnp.dot(xb, wb[...], preferred_element_type=jnp.float32) + bb[...]

    return pl.pallas_call(
        body,
        grid=(nb,),
        in_specs=[
            pl.BlockSpec((_BN, d), lambda i: (i, 0)),
            pl.BlockSpec(WnA.shape, lambda i: (0, 0)),
            pl.BlockSpec(WnB.shape, lambda i: (0, 0)),
            pl.BlockSpec((1, _F), lambda i: (0, 0)),
            pl.BlockSpec((1, _F), lambda i: (0, 0)),
        ],
        out_specs=[
            pl.BlockSpec((_BN, _F), lambda i: (i, 0)),
            pl.BlockSpec((_BN, _F), lambda i: (i, 0)),
        ],
        out_shape=[
            jax.ShapeDtypeStruct((n, _F), jnp.float32),
            jax.ShapeDtypeStruct((n, _F), jnp.float32),
        ],
    )(x, WnA, WnB, bnA, bnB)


def _embed_edges(edge_attr, WeA, WeB, beA, beB):
    e, d = edge_attr.shape
    nb = e // _BE

    def body(a_ref, wa, wb, ba, bb, oa, ob):
        a = a_ref[...]
        oa[...] = (ba[...] + a[:, 0:1] * wa[0:1, :] + a[:, 1:2] * wa[1:2, :]
                   + a[:, 2:3] * wa[2:3, :])
        ob[...] = (bb[...] + a[:, 0:1] * wb[0:1, :] + a[:, 1:2] * wb[1:2, :]
                   + a[:, 2:3] * wb[2:3, :])

    return pl.pallas_call(
        body,
        grid=(nb,),
        in_specs=[
            pl.BlockSpec((_BE, d), lambda i: (i, 0)),
            pl.BlockSpec(WeA.shape, lambda i: (0, 0)),
            pl.BlockSpec(WeB.shape, lambda i: (0, 0)),
            pl.BlockSpec((1, _F), lambda i: (0, 0)),
            pl.BlockSpec((1, _F), lambda i: (0, 0)),
        ],
        out_specs=[
            pl.BlockSpec((_BE, _F), lambda i: (i, 0)),
            pl.BlockSpec((_BE, _F), lambda i: (i, 0)),
        ],
        out_shape=[
            jax.ShapeDtypeStruct((e, _F), jnp.float32),
            jax.ShapeDtypeStruct((e, _F), jnp.float32),
        ],
    )(edge_attr, WeA, WeB, beA, beB)


def _sc_conv(hA, hB, eaA, eaB, src, dst):
    n = hA.shape[0]
    e = src.shape[0]
    ept = e // _NS
    nblk = ept // _EBLK
    zchunks = n // _ZR
    nwb = n // _WB
    mesh = plsc.VectorSubcoreMesh(core_axis_name="c", subcore_axis_name="s",
                                  num_cores=_NC, num_subcores=_NS)

    @functools.partial(
        pl.kernel,
        out_type=[
            jax.ShapeDtypeStruct((n, _F), jnp.float32),
            jax.ShapeDtypeStruct((n, _F), jnp.float32),
        ],
        mesh=mesh,
        compiler_params=pltpu.CompilerParams(use_tc_tiling_on_sc=False),
        scratch_types=[
            pltpu.VMEM((_EBLK,), jnp.int32),
            pltpu.VMEM((_EBLK,), jnp.int32),
            pltpu.VMEM((_EBLK, _F), jnp.float32),
            pltpu.VMEM((_EBLK, _F), jnp.float32),
            pltpu.VMEM((_ZR, _F), jnp.float32),
            pltpu.SemaphoreType.DMA,
            pltpu.SemaphoreType.DMA,
            pltpu.VMEM_SHARED((n, _F), jnp.float32),
        ],
    )
    def conv(hA_hbm, hB_hbm, eaA_hbm, eaB_hbm, src_hbm, dst_hbm,
             outA_hbm, outB_hbm,
             sidx, didx, hbuf, ebuf, zbuf, gsem, esem, acc):
        c = lax.axis_index("c")
        s = lax.axis_index("s")
        zv = jnp.zeros((_F,), jnp.float32)

        def zrow(i, carry):
            zbuf[i, :] = zv
            return carry

        lax.fori_loop(0, _ZR, zrow, 0, unroll=8)

        def zcopy(k, carry):
            j = s + k * _NS

            @pl.when(j < zchunks)
            def _():
                pltpu.sync_copy(zbuf, acc.at[pl.ds(j * _ZR, _ZR)])

            return carry

        lax.fori_loop(0, (zchunks + _NS - 1) // _NS, zcopy, 0)
        plsc.subcore_barrier()

        def run(h_hbm, ea_hbm):
            def blk(b, carry):
                base = s * ept + b * _EBLK
                pltpu.sync_copy(src_hbm.at[pl.ds(base, _EBLK)], sidx)
                pltpu.sync_copy(dst_hbm.at[pl.ds(base, _EBLK)], didx)
                cea = pltpu.async_copy(ea_hbm.at[pl.ds(base, _EBLK)], ebuf, esem)
                cg = pltpu.async_copy(h_hbm.at[sidx], hbuf, gsem)
                cea.wait()
                cg.wait()

                def row(i, carry2):
                    hbuf[i, :] = jnp.maximum(hbuf[i, :] + ebuf[i, :], 0.0)
                    return carry2

                lax.fori_loop(0, _EBLK, row, 0, unroll=8)
                pltpu.sync_copy(hbuf, acc.at[didx], add=True)
                return carry

            lax.fori_loop(0, nblk, blk, 0)

        @pl.when(c == 0)
        def _():
            run(hA_hbm, eaA_hbm)

        @pl.when(c == 1)
        def _():
            run(hB_hbm, eaB_hbm)

        plsc.subcore_barrier()

        def wb(out_hbm):
            def wcopy(k, carry):
                j = s + k * _NS

                @pl.when(j < nwb)
                def _():
                    pltpu.sync_copy(acc.at[pl.ds(j * _WB, _WB)],
                                    out_hbm.at[pl.ds(j * _WB, _WB)])

                return carry

            lax.fori_loop(0, (nwb + _NS - 1) // _NS, wcopy, 0)

        @pl.when(c == 0)
        def _():
            wb(outA_hbm)

        @pl.when(c == 1)
        def _():
            wb(outB_hbm)

    return conv(hA, hB, eaA, eaB, src, dst)


def _mlp_stats(hA, hB, agA, agB, W1A, W1B, b1, W2, b2):
    n = hA.shape[0]
    nb = n // _BN

    def body(ha, hb, aa, ab, w1a, w1b, b1r, w2, b2r, z_ref, s1_ref, s2_ref):
        i = pl.program_id(0)
        xa = ha[...] + aa[...]
        xb = hb[...] + ab[...]
        t = jnp.maximum(
            jnp.dot(xa, w1a[...], preferred_element_type=jnp.float32)
            + jnp.dot(xb, w1b[...], preferred_element_type=jnp.float32)
            + b1r[...], 0.0)
        z = jnp.dot(t, w2[...], preferred_element_type=jnp.float32) + b2r[...]
        z_ref[...] = z
        p1 = jnp.sum(z, axis=0, keepdims=True)
        p2 = jnp.sum(z * z, axis=0, keepdims=True)

        @pl.when(i == 0)
        def _():
            s1_ref[...] = p1
            s2_ref[...] = p2

        @pl.when(i > 0)
        def _():
            s1_ref[...] += p1
            s2_ref[...] += p2

    return pl.pallas_call(
        body,
        grid=(nb,),
        in_specs=[
            pl.BlockSpec((_BN, _F), lambda i: (i, 0)),
            pl.BlockSpec((_BN, _F), lambda i: (i, 0)),
            pl.BlockSpec((_BN, _F), lambda i: (i, 0)),
            pl.BlockSpec((_BN, _F), lambda i: (i, 0)),
            pl.BlockSpec(W1A.shape, lambda i: (0, 0)),
            pl.BlockSpec(W1B.shape, lambda i: (0, 0)),
            pl.BlockSpec(b1.shape, lambda i: (0, 0)),
            pl.BlockSpec(W2.shape, lambda i: (0, 0)),
            pl.BlockSpec(b2.shape, lambda i: (0, 0)),
        ],
        out_specs=[
            pl.BlockSpec((_BN, 2 * _F), lambda i: (i, 0)),
            pl.BlockSpec((1, 2 * _F), lambda i: (0, 0)),
            pl.BlockSpec((1, 2 * _F), lambda i: (0, 0)),
        ],
        out_shape=[
            jax.ShapeDtypeStruct((n, 2 * _F), jnp.float32),
            jax.ShapeDtypeStruct((1, 2 * _F), jnp.float32),
            jax.ShapeDtypeStruct((1, 2 * _F), jnp.float32),
        ],
    )(hA, hB, agA, agB, W1A, W1B, b1, W2, b2)


def _norm_relu(z, s1, s2, g, beta):
    n = z.shape[0]
    nb = n // _BN
    inv_n = 1.0 / n

    def body(z_ref, s1_ref, s2_ref, g_ref, b_ref, oa, ob):
        mu = s1_ref[...] * inv_n
        var = s2_ref[...] * inv_n - mu * mu
        scale = g_ref[...] * lax.rsqrt(var + 1e-5)
        h = jnp.maximum((z_ref[...] - mu) * scale + b_ref[...], 0.0)
        oa[...] = h[:, :_F]
        ob[...] = h[:, _F:]

    return pl.pallas_call(
        body,
        grid=(nb,),
        in_specs=[
            pl.BlockSpec((_BN, 2 * _F), lambda i: (i, 0)),
            pl.BlockSpec((1, 2 * _F), lambda i: (0, 0)),
            pl.BlockSpec((1, 2 * _F), lambda i: (0, 0)),
            pl.BlockSpec((1, 2 * _F), lambda i: (0, 0)),
            pl.BlockSpec((1, 2 * _F), lambda i: (0, 0)),
        ],
        out_specs=[
            pl.BlockSpec((_BN, _F), lambda i: (i, 0)),
            pl.BlockSpec((_BN, _F), lambda i: (i, 0)),
        ],
        out_shape=[
            jax.ShapeDtypeStruct((n, _F), jnp.float32),
            jax.ShapeDtypeStruct((n, _F), jnp.float32),
        ],
    )(z, s1, s2, g, beta)


def _pool(hA, hB, batch2d):
    n = hA.shape[0]
    nb = n // _BN

    def body(ha, hb, bt, sa_ref, sb_ref, c_ref):
        i = pl.program_id(0)
        ids = bt[...]
        lab = lax.broadcasted_iota(jnp.int32, (_BN, _G), 1)
        oh = (ids == lab).astype(jnp.float32)
        dn = (((0,), (0,)), ((), ()))
        pa = lax.dot_general(oh, ha[...], dn, preferred_element_type=jnp.float32)
        pb = lax.dot_general(oh, hb[...], dn, preferred_element_type=jnp.float32)
        ones = jnp.ones((_BN, 1), jnp.float32)
        pc = lax.dot_general(oh, ones, dn, preferred_element_type=jnp.float32)

        @pl.when(i == 0)
        def _():
            sa_ref[...] = pa
            sb_ref[...] = pb
            c_ref[...] = pc

        @pl.when(i > 0)
        def _():
            sa_ref[...] += pa
            sb_ref[...] += pb
            c_ref[...] += pc

    return pl.pallas_call(
        body,
        grid=(nb,),
        in_specs=[
            pl.BlockSpec((_BN, _F), lambda i: (i, 0)),
            pl.BlockSpec((_BN, _F), lambda i: (i, 0)),
            pl.BlockSpec((_BN, 1), lambda i: (i, 0)),
        ],
        out_specs=[
            pl.BlockSpec((_G, _F), lambda i: (0, 0)),
            pl.BlockSpec((_G, _F), lambda i: (0, 0)),
            pl.BlockSpec((_G, 1), lambda i: (0, 0)),
        ],
        out_shape=[
            jax.ShapeDtypeStruct((_G, _F), jnp.float32),
            jax.ShapeDtypeStruct((_G, _F), jnp.float32),
            jax.ShapeDtypeStruct((_G, 1), jnp.float32),
        ],
    )(hA, hB, batch2d)


def _head(sA, sB, cnt, Wl1A, Wl1B, bl1, Wl2, bl2):
    def body(sa, sb, c, w1a, w1b, b1r, w2, b2r, o_ref):
        inv = 1.0 / jnp.maximum(c[...], 1.0)
        ga = sa[...] * inv
        gb = sb[...] * inv
        t = jnp.maximum(
            jnp.dot(ga, w1a[...], preferred_element_type=jnp.float32)
            + jnp.dot(gb, w1b[...], preferred_element_type=jnp.float32)
            + b1r[...], 0.0)
        o_ref[...] = jnp.dot(t, w2[...], preferred_element_type=jnp.float32) + b2r[...]

    return pl.pallas_call(
        body,
        out_shape=jax.ShapeDtypeStruct((_G, 2), jnp.float32),
    )(sA, sB, cnt, Wl1A, Wl1B, bl1, Wl2, bl2)


def kernel(x, edge_index, edge_attr, batch, Wn, bn, We, be,
           W1_0, b1_0, W2_0, b2_0, g0, beta0,
           W1_1, b1_1, W2_1, b2_1, g1, beta1,
           Wl1, bl1, Wl2, bl2):
    src = edge_index[0]
    dst = edge_index[1]
    batch2d = batch.reshape(-1, 1)

    hA, hB = _embed_nodes(x, Wn[:, :_F], Wn[:, _F:],
                          bn[:_F].reshape(1, _F), bn[_F:].reshape(1, _F))
    eaA, eaB = _embed_edges(edge_attr, We[:, :_F], We[:, _F:],
                            be[:_F].reshape(1, _F), be[_F:].reshape(1, _F))

    agA, agB = _sc_conv(hA, hB, eaA, eaB, src, dst)
    z, s1, s2 = _mlp_stats(hA, hB, agA, agB, W1_0[:_F], W1_0[_F:],
                           b1_0.reshape(1, -1), W2_0, b2_0.reshape(1, -1))
    h1A, h1B = _norm_relu(z, s1, s2, g0.reshape(1, -1), beta0.reshape(1, -1))

    agA, agB = _sc_conv(h1A, h1B, eaA, eaB, src, dst)
    z, s1, s2 = _mlp_stats(h1A, h1B, agA, agB, W1_1[:_F], W1_1[_F:],
                           b1_1.reshape(1, -1), W2_1, b2_1.reshape(1, -1))
    h2A, h2B = _norm_relu(z, s1, s2, g1.reshape(1, -1), beta1.reshape(1, -1))

    sA, sB, cnt = _pool(h2A, h2B, batch2d)
    return _head(sA, sB, cnt, Wl1[:_F], Wl1[_F:],
                 bl1.reshape(1, -1), Wl2, bl2.reshape(1, -1))
